# baseline (speedup 1.0000x reference)
import functools

import jax
import jax.numpy as jnp
from jax import lax
from jax.experimental import pallas as pl
from jax.experimental.pallas import tpu as pltpu
from jax.experimental.pallas import tpu_sc as plsc


def _build_emb_kernel(N, V, DE, DG, DGP, NC, per_w, NB, NL):
    n_chunks = per_w // NB
    DGM = DG - 4
    mesh = plsc.VectorSubcoreMesh(core_axis_name="c", subcore_axis_name="s")

    @functools.partial(
        pl.kernel,
        mesh=mesh,
        out_type=jax.ShapeDtypeStruct((N, DE + DG), jnp.float32),
        scratch_types=[
            pltpu.VMEM((NB,), jnp.int32),
            pltpu.VMEM((NB, DE), jnp.float32),
            pltpu.VMEM((NB, DGP), jnp.float32),
            pltpu.VMEM((NB, 4), jnp.float32),
            pltpu.SemaphoreType.DMA,
            pltpu.SemaphoreType.DMA,
        ],
        compiler_params=pltpu.CompilerParams(
            use_tc_tiling_on_sc=False, needs_layout_passes=False),
    )
    def emb_kernel(idx_hbm, elmo_hbm, glove_hbm, out_hbm,
                   idx_v, erows_v, grows_v, tail_v, esem, gsem):
        wid = lax.axis_index("s") * NC + lax.axis_index("c")
        base_w = wid * per_w
        lane = lax.iota(jnp.int32, NL)

        def body(c, carry):
            base = base_w + c * NB
            pltpu.sync_copy(idx_hbm.at[pl.ds(base, NB)], idx_v)
            ec = pltpu.async_copy(elmo_hbm.at[idx_v], erows_v, esem)
            gc = pltpu.async_copy(glove_hbm.at[idx_v], grows_v, gsem)
            ec.wait()
            pltpu.sync_copy(erows_v, out_hbm.at[pl.ds(base, NB), pl.ds(0, DE)])
            gc.wait()
            pltpu.sync_copy(grows_v.at[:, pl.ds(0, DGM)],
                            out_hbm.at[pl.ds(base, NB), pl.ds(DE, DGM)])
            for t in range(NB * 4 // NL):
                e = t * NL + lane
                rows = lax.shift_right_logical(e, 2)
                cols = lax.bitwise_and(e, 3)
                vals = plsc.load_gather(grows_v, [rows, cols + DGM])
                plsc.store_scatter(tail_v, [rows, cols], vals)
            pltpu.sync_copy(tail_v,
                            out_hbm.at[pl.ds(base, NB), pl.ds(DE + DGM, 4)])
            return carry

        lax.fori_loop(0, n_chunks, body, 0)

    return emb_kernel


def kernel(sentences, lengths, elmo_table, glove_table):
    B, L = sentences.shape
    V, DE = elmo_table.shape
    DG = glove_table.shape[1]
    N = B * L

    info = plsc.get_sparse_core_info()
    NC, NS, NL = info.num_cores, info.num_subcores, info.num_lanes
    NW = NC * NS
    per_w = N // NW
    NB = 64

    DGP = DG + 4
    idx = sentences.reshape(N).astype(jnp.int32)
    glove_p = jnp.pad(glove_table, ((0, 0), (0, DGP - DG)))
    emb = _build_emb_kernel(N, V, DE, DG, DGP, NC, per_w, NB, NL)
    out = emb(idx, elmo_table, glove_p)
    return out.reshape(B, L, DE + DG)


# speedup vs baseline: 2.1902x; 2.1902x over previous
import functools

import jax
import jax.numpy as jnp
from jax import lax
from jax.experimental import pallas as pl
from jax.experimental.pallas import tpu as pltpu
from jax.experimental.pallas import tpu_sc as plsc


def _build_emb_kernel(N, V, DE, DG, NC, per_w, NB, NL):
    n_chunks = per_w // NB
    DGA = 256
    DGB = DG - DGA
    TW = 128
    mesh = plsc.VectorSubcoreMesh(core_axis_name="c", subcore_axis_name="s")

    @functools.partial(
        pl.kernel,
        mesh=mesh,
        out_type=jax.ShapeDtypeStruct((N, DE + DG), jnp.float32),
        scratch_types=[
            pltpu.VMEM((NB,), jnp.int32),
            pltpu.VMEM((NB, DE), jnp.float32),
            pltpu.VMEM((NB, DGA), jnp.float32),
            pltpu.VMEM((NB, TW), jnp.float32),
            pltpu.VMEM((NB, DGB), jnp.float32),
            pltpu.SemaphoreType.DMA,
            pltpu.SemaphoreType.DMA,
            pltpu.SemaphoreType.DMA,
        ],
        compiler_params=pltpu.CompilerParams(
            use_tc_tiling_on_sc=True, needs_layout_passes=False),
    )
    def emb_kernel(idx_hbm, elmo_hbm, glove_hbm, tail_hbm, out_hbm,
                   idx_v, erows_v, ga_v, t128_v, gb_v, esem, gsem, tsem):
        wid = lax.axis_index("s") * NC + lax.axis_index("c")
        base_w = wid * per_w
        lane = lax.iota(jnp.int32, NL)

        def body(c, carry):
            base = base_w + c * NB
            pltpu.sync_copy(idx_hbm.at[pl.ds(base, NB)], idx_v)
            ec = pltpu.async_copy(elmo_hbm.at[idx_v], erows_v, esem)
            gc = pltpu.async_copy(glove_hbm.at[idx_v, pl.ds(0, DGA)], ga_v, gsem)
            tc = pltpu.async_copy(tail_hbm.at[idx_v], t128_v, tsem)
            ec.wait()
            pltpu.sync_copy(erows_v, out_hbm.at[pl.ds(base, NB), pl.ds(0, DE)])
            gc.wait()
            pltpu.sync_copy(ga_v, out_hbm.at[pl.ds(base, NB), pl.ds(DE, DGA)])
            tc.wait()
            for t in range(NB * DGB // NL):
                e = t * NL + lane
                rows = e // DGB
                cols = e % DGB
                vals = plsc.load_gather(t128_v, [rows, cols + (TW - DGB)])
                plsc.store_scatter(gb_v, [rows, cols], vals)
            pltpu.sync_copy(gb_v,
                            out_hbm.at[pl.ds(base, NB), pl.ds(DE + DGA, DGB)])
            return carry

        lax.fori_loop(0, n_chunks, body, 0)

    return emb_kernel


def kernel(sentences, lengths, elmo_table, glove_table):
    B, L = sentences.shape
    V, DE = elmo_table.shape
    DG = glove_table.shape[1]
    N = B * L

    info = plsc.get_sparse_core_info()
    NC, NS, NL = info.num_cores, info.num_subcores, info.num_lanes
    NW = NC * NS
    per_w = N // NW
    NB = 64

    idx = sentences.reshape(N).astype(jnp.int32)
    tail_t = lax.slice(glove_table, (0, DG - 128), (V, DG))
    emb = _build_emb_kernel(N, V, DE, DG, NC, per_w, NB, NL)
    out = emb(idx, elmo_table, glove_table, tail_t)
    return out.reshape(B, L, DE + DG)
